# baseline (speedup 1.0000x reference)
import functools

import jax
import jax.numpy as jnp
from jax import lax
from jax.experimental import pallas as pl
from jax.experimental.pallas import tpu as pltpu
from jax.experimental.pallas import tpu_sc as plsc

NC = 2
NS = 16
L = 16
NW = NC * NS
B = 16384
D = 32
BPW = B // NW
NCH = 4
CH = BPW // NCH
NSLOT = 8
TW = 128

_mesh = plsc.VectorSubcoreMesh(core_axis_name="c", subcore_axis_name="s")


@functools.partial(
    pl.kernel,
    out_type=jax.ShapeDtypeStruct((NW, NCH, CH), jnp.float32),
    mesh=_mesh,
    compiler_params=pltpu.CompilerParams(needs_layout_passes=False),
    scratch_types=[
        pltpu.VMEM((NCH, CH), jnp.int32),
        pltpu.VMEM((NCH, CH), jnp.int32),
        pltpu.VMEM((NSLOT, D, TW), jnp.float32),
        pltpu.VMEM((NSLOT, D, TW), jnp.float32),
        pltpu.VMEM((NCH, CH), jnp.float32),
        pltpu.VMEM((NCH, CH), jnp.float32),
        pltpu.VMEM((L,), jnp.float32),
        pltpu.VMEM((NCH, CH), jnp.float32),
        pltpu.SemaphoreType.DMA,
        pltpu.SemaphoreType.DMA,
    ],
)
def _mf_sc(uid_hbm, iid_hbm, uembt_hbm, iembt_hbm, ubias_hbm, ibias_hbm,
           gb_hbm, out_hbm, uid_v, iid_v, ubuf, ibuf, bu_v, bi_v,
           gb_v, out_v, semd, semb):
    wid = lax.axis_index("s") * NC + lax.axis_index("c")

    pltpu.sync_copy(uid_hbm.at[wid], uid_v)
    pltpu.sync_copy(iid_hbm.at[wid], iid_v)
    pltpu.sync_copy(gb_hbm, gb_v)

    bias_copies = []
    for c in range(NCH):
        bias_copies.append(
            pltpu.async_copy(ubias_hbm.at[uid_v.at[c]], bu_v.at[c], semb))
        bias_copies.append(
            pltpu.async_copy(ibias_hbm.at[iid_v.at[c]], bi_v.at[c], semb))

    gb = gb_v[:]
    lane = lax.iota(jnp.int32, L)
    rows0 = lax.iota(jnp.int32, L)
    rows1 = rows0 + L

    def fire(uo, io, j):
        for a in range(D // 8):
            pltpu.async_copy(
                uembt_hbm.at[pl.ds(a * 8, 8), pl.ds(uo, TW)],
                ubuf.at[j, pl.ds(a * 8, 8)], semd)
            pltpu.async_copy(
                iembt_hbm.at[pl.ds(a * 8, 8), pl.ds(io, TW)],
                ibuf.at[j, pl.ds(a * 8, 8)], semd)

    def drain_pair():
        for a in range(2 * (D // 8)):
            pltpu.make_async_copy(
                uembt_hbm.at[pl.ds(0, 8), pl.ds(0, TW)],
                ubuf.at[0, pl.ds(0, 8)], semd).wait()

    uvec0 = uid_v[0, pl.ds(0, L)]
    ivec0 = iid_v[0, pl.ds(0, L)]
    for j in range(NSLOT):
        fire((uvec0[j] >> 7) * TW, (ivec0[j] >> 7) * TW, j)

    NG = CH // L

    for c in range(NCH):
        def group_body(g, carry, c=c):
            base = g * L
            uvec = uid_v[c, pl.ds(base, L)]
            ivec = iid_v[c, pl.ds(base, L)]
            ucol = uvec & (TW - 1)
            icol = ivec & (TW - 1)
            nbase = jnp.where(g + 1 < NG, base + L, 0)
            nc = c + 1 if c + 1 < NCH else c
            if c + 1 < NCH:
                uvec_n = jnp.where(g + 1 < NG,
                                   uid_v[c, pl.ds(nbase, L)],
                                   uid_v[nc, pl.ds(0, L)])
                ivec_n = jnp.where(g + 1 < NG,
                                   iid_v[c, pl.ds(nbase, L)],
                                   iid_v[nc, pl.ds(0, L)])
            else:
                uvec_n = uid_v[c, pl.ds(nbase, L)]
                ivec_n = iid_v[c, pl.ds(nbase, L)]
            acc = jnp.zeros((L,), jnp.float32)
            for h in range(2):
                for j in range(NSLOT):
                    jj = h * NSLOT + j
                    drain_pair()
                    cu = jnp.broadcast_to(ucol[jj], (L,))
                    ci = jnp.broadcast_to(icol[jj], (L,))
                    u0 = plsc.load_gather(ubuf.at[j], [rows0, cu])
                    u1 = plsc.load_gather(ubuf.at[j], [rows1, cu])
                    i0 = plsc.load_gather(ibuf.at[j], [rows0, ci])
                    i1 = plsc.load_gather(ibuf.at[j], [rows1, ci])
                    s = jnp.sum(u0 * i0 + u1 * i1)
                    acc = jnp.where(lane == jj, s, acc)
                    if h == 0:
                        nu, ni = uvec[jj + NSLOT], ivec[jj + NSLOT]
                    else:
                        nu, ni = uvec_n[j], ivec_n[j]
                    fire((nu >> 7) * TW, (ni >> 7) * TW, j)
            out_v[c, pl.ds(base, L)] = acc
            return carry
        lax.fori_loop(0, NG, group_body, 0)

    for j in range(NSLOT):
        drain_pair()

    for cp in bias_copies:
        cp.wait()
    for c in range(NCH):
        def bias_body(g, carry, c=c):
            base = g * L
            out_v[c, pl.ds(base, L)] = (
                out_v[c, pl.ds(base, L)] + bu_v[c, pl.ds(base, L)]
                + bi_v[c, pl.ds(base, L)] + gb
            )
            return carry
        lax.fori_loop(0, CH // L, bias_body, 0)

    pltpu.sync_copy(out_v, out_hbm.at[wid])


def kernel(user_id, item_id, user_embedding, item_embedding, user_bias,
           item_bias, global_bias):
    uid = user_id.astype(jnp.int32).reshape(NW, NCH, CH)
    iid = item_id.astype(jnp.int32).reshape(NW, NCH, CH)
    gb16 = jnp.broadcast_to(global_bias.astype(jnp.float32), (L,))
    out = _mf_sc(uid, iid, user_embedding.T, item_embedding.T, user_bias,
                 item_bias, gb16)
    return out.reshape(B)


# speedup vs baseline: 1.0275x; 1.0275x over previous
import functools

import jax
import jax.numpy as jnp
from jax import lax
from jax.experimental import pallas as pl
from jax.experimental.pallas import tpu as pltpu
from jax.experimental.pallas import tpu_sc as plsc

NC = 2
NS = 16
L = 16
NW = NC * NS
B = 16384
D = 32
BPW = B // NW
NCH = 4
CH = BPW // NCH
NSLOT = 8
TW = 128

_mesh = plsc.VectorSubcoreMesh(core_axis_name="c", subcore_axis_name="s")


@functools.partial(
    pl.kernel,
    out_type=jax.ShapeDtypeStruct((NW, NCH, CH), jnp.float32),
    mesh=_mesh,
    compiler_params=pltpu.CompilerParams(needs_layout_passes=False),
    scratch_types=[
        pltpu.VMEM((NCH, CH), jnp.int32),
        pltpu.VMEM((NCH, CH), jnp.int32),
        pltpu.VMEM((NSLOT, D, TW), jnp.float32),
        pltpu.VMEM((NSLOT, D, TW), jnp.float32),
        pltpu.VMEM((NCH, CH), jnp.float32),
        pltpu.VMEM((NCH, CH), jnp.float32),
        pltpu.VMEM((L,), jnp.float32),
        pltpu.VMEM((NCH, CH), jnp.float32),
        pltpu.SemaphoreType.DMA,
        pltpu.SemaphoreType.DMA,
    ],
)
def _mf_sc(uid_hbm, iid_hbm, uembt_hbm, iembt_hbm, ubias_hbm, ibias_hbm,
           gb_hbm, out_hbm, uid_v, iid_v, ubuf, ibuf, bu_v, bi_v,
           gb_v, out_v, semd, semb):
    wid = lax.axis_index("s") * NC + lax.axis_index("c")

    pltpu.sync_copy(uid_hbm.at[wid], uid_v)
    pltpu.sync_copy(iid_hbm.at[wid], iid_v)
    pltpu.sync_copy(gb_hbm, gb_v)

    bias_copies = []
    for c in range(NCH):
        bias_copies.append(
            pltpu.async_copy(ubias_hbm.at[uid_v.at[c]], bu_v.at[c], semb))
        bias_copies.append(
            pltpu.async_copy(ibias_hbm.at[iid_v.at[c]], bi_v.at[c], semb))

    gb = gb_v[:]
    lane = lax.iota(jnp.int32, L)
    rows0 = lax.iota(jnp.int32, L)
    rows1 = rows0 + L

    def fire(uo, io, j):
        pltpu.async_copy(uembt_hbm.at[:, pl.ds(uo, TW)], ubuf.at[j], semd)
        pltpu.async_copy(iembt_hbm.at[:, pl.ds(io, TW)], ibuf.at[j], semd)

    def drain_pair():
        pltpu.make_async_copy(
            uembt_hbm.at[:, pl.ds(0, TW)], ubuf.at[0], semd).wait()
        pltpu.make_async_copy(
            uembt_hbm.at[:, pl.ds(0, TW)], ubuf.at[0], semd).wait()

    uvec0 = uid_v[0, pl.ds(0, L)]
    ivec0 = iid_v[0, pl.ds(0, L)]
    for j in range(NSLOT):
        fire((uvec0[j] >> 7) * TW, (ivec0[j] >> 7) * TW, j)

    NG = CH // L

    for c in range(NCH):
        def group_body(g, carry, c=c):
            base = g * L
            uvec = uid_v[c, pl.ds(base, L)]
            ivec = iid_v[c, pl.ds(base, L)]
            ucol = uvec & (TW - 1)
            icol = ivec & (TW - 1)
            nbase = jnp.where(g + 1 < NG, base + L, 0)
            nc = c + 1 if c + 1 < NCH else c
            if c + 1 < NCH:
                uvec_n = jnp.where(g + 1 < NG,
                                   uid_v[c, pl.ds(nbase, L)],
                                   uid_v[nc, pl.ds(0, L)])
                ivec_n = jnp.where(g + 1 < NG,
                                   iid_v[c, pl.ds(nbase, L)],
                                   iid_v[nc, pl.ds(0, L)])
            else:
                uvec_n = uid_v[c, pl.ds(nbase, L)]
                ivec_n = iid_v[c, pl.ds(nbase, L)]
            acc = jnp.zeros((L,), jnp.float32)
            for h in range(2):
                for j in range(NSLOT):
                    jj = h * NSLOT + j
                    drain_pair()
                    cu = jnp.broadcast_to(ucol[jj], (L,))
                    ci = jnp.broadcast_to(icol[jj], (L,))
                    u0 = plsc.load_gather(ubuf.at[j], [rows0, cu])
                    u1 = plsc.load_gather(ubuf.at[j], [rows1, cu])
                    i0 = plsc.load_gather(ibuf.at[j], [rows0, ci])
                    i1 = plsc.load_gather(ibuf.at[j], [rows1, ci])
                    s = jnp.sum(u0 * i0 + u1 * i1)
                    acc = jnp.where(lane == jj, s, acc)
                    if h == 0:
                        nu, ni = uvec[jj + NSLOT], ivec[jj + NSLOT]
                    else:
                        nu, ni = uvec_n[j], ivec_n[j]
                    fire((nu >> 7) * TW, (ni >> 7) * TW, j)
            out_v[c, pl.ds(base, L)] = acc
            return carry
        lax.fori_loop(0, NG, group_body, 0)

    for j in range(NSLOT):
        drain_pair()

    for cp in bias_copies:
        cp.wait()
    for c in range(NCH):
        def bias_body(g, carry, c=c):
            base = g * L
            out_v[c, pl.ds(base, L)] = (
                out_v[c, pl.ds(base, L)] + bu_v[c, pl.ds(base, L)]
                + bi_v[c, pl.ds(base, L)] + gb
            )
            return carry
        lax.fori_loop(0, CH // L, bias_body, 0)

    pltpu.sync_copy(out_v, out_hbm.at[wid])


def kernel(user_id, item_id, user_embedding, item_embedding, user_bias,
           item_bias, global_bias):
    uid = user_id.astype(jnp.int32).reshape(NW, NCH, CH)
    iid = item_id.astype(jnp.int32).reshape(NW, NCH, CH)
    gb16 = jnp.broadcast_to(global_bias.astype(jnp.float32), (L,))
    out = _mf_sc(uid, iid, user_embedding.T, item_embedding.T, user_bias,
                 item_bias, gb16)
    return out.reshape(B)
